# baseline (speedup 1.0000x reference)
import functools

import jax
import jax.numpy as jnp
from jax import lax
from jax.experimental import pallas as pl
from jax.experimental.pallas import tpu as pltpu
from jax.experimental.pallas import tpu_sc as plsc

_NUM_POINTS = 32
_CROP = 224
_THRESHOLD = 0.15
_HALF_ROWS = _CROP // 2


def _sample_yx(salience_map):
    H, W = salience_map.shape
    prob = salience_map.reshape(-1)
    y_t = max(_CROP // 2, int(_THRESHOLD * H))
    x_t = max(_CROP // 2, int(_THRESHOLD * H))
    border_mask = jnp.zeros((H, W), dtype=salience_map.dtype)
    border_mask = border_mask.at[y_t:H - y_t, x_t:W - x_t].set(1.0)
    border_mask = border_mask.reshape(-1)
    p = prob * border_mask
    p = p / p.sum()
    p = jax.lax.stop_gradient(p)
    skey = jax.random.key(42)
    idx = jax.random.choice(skey, prob.shape[0], shape=(_NUM_POINTS,),
                            replace=True, p=p)
    y = idx // W
    x = idx % W
    return y, x


_SC_MESH = plsc.VectorSubcoreMesh(core_axis_name="c", subcore_axis_name="s")


@functools.partial(
    pl.kernel,
    mesh=_SC_MESH,
    out_type=jax.ShapeDtypeStruct((_NUM_POINTS * 3 * _CROP, _CROP),
                                  jnp.float32),
    scratch_types=[
        [pltpu.VMEM((_HALF_ROWS,), jnp.int32) for _ in range(6)],
        pltpu.VMEM((_HALF_ROWS, 512), jnp.float32),
        pltpu.VMEM((_HALF_ROWS, _CROP), jnp.float32),
        pltpu.VMEM((16,), jnp.int32),
        pltpu.SemaphoreType.DMA,
    ],
    compiler_params=pltpu.CompilerParams(needs_layout_passes=False),
)
def _sc_crop_kernel(rows_hbm, idx_hbm, left_hbm, out_hbm,
                    idx_vs, g_v, l_v, lw_v, sem):
    wid = lax.axis_index("s") * 2 + lax.axis_index("c")

    for u in range(6):
        pltpu.sync_copy(
            idx_hbm.at[pl.ds((wid * 6 + u) * _HALF_ROWS, _HALF_ROWS)],
            idx_vs[u])
    pltpu.sync_copy(left_hbm.at[pl.ds(wid * 16, 16)], lw_v)
    lane = lax.iota(jnp.int32, 16)
    lv = lax.reduce_sum_p.bind(
        jnp.where(lane == 0, lw_v[...], 0), axes=(0,))
    src0 = lane + lv

    for c in range(3):
        for h in range(2):
            pltpu.async_copy(rows_hbm.at[idx_vs[c * 2 + h]], g_v, sem).wait()

            def body(i, carry):
                row_idx = jnp.full((16,), i, dtype=jnp.int32)
                for j in range(14):
                    chunk = plsc.load_gather(g_v, [row_idx, src0 + 16 * j])
                    plsc.store_scatter(l_v, [row_idx, lane + 16 * j], chunk)
                return carry

            lax.fori_loop(0, _HALF_ROWS, body, 0)
            pltpu.sync_copy(
                l_v,
                out_hbm.at[pl.ds((wid * 3 + c) * _CROP + h * _HALF_ROWS,
                                 _HALF_ROWS), :])


def kernel(img, salience_map):
    y, x = _sample_yx(salience_map)
    half = _CROP // 2
    top = (y - half).astype(jnp.int32)
    left = (x - half).astype(jnp.int32)
    C, H, W = img.shape

    rows = img.reshape(C * H, W)
    cc = jnp.arange(3, dtype=jnp.int32)[None, :, None, None] * H
    hh = jnp.arange(2, dtype=jnp.int32)[None, None, :, None] * _HALF_ROWS
    ii = jnp.arange(_HALF_ROWS, dtype=jnp.int32)[None, None, None, :]
    idx = (top[:, None, None, None] + cc + hh + ii).reshape(-1)
    lpad = jnp.zeros((_NUM_POINTS, 16), jnp.int32).at[:, 0].set(left)

    out = _sc_crop_kernel(rows, idx, lpad.reshape(-1))
    return out.reshape(_NUM_POINTS, C, _CROP, _CROP)


# speedup vs baseline: 1.2067x; 1.2067x over previous
import functools

import jax
import jax.numpy as jnp
from jax import lax
from jax.experimental import pallas as pl
from jax.experimental.pallas import tpu as pltpu
from jax.experimental.pallas import tpu_sc as plsc

_NUM_POINTS = 32
_CROP = 224
_THRESHOLD = 0.15
_HALF_ROWS = _CROP // 2


def _sample_yx(salience_map):
    H, W = salience_map.shape
    prob = salience_map.reshape(-1)
    y_t = max(_CROP // 2, int(_THRESHOLD * H))
    x_t = max(_CROP // 2, int(_THRESHOLD * H))
    border_mask = jnp.zeros((H, W), dtype=salience_map.dtype)
    border_mask = border_mask.at[y_t:H - y_t, x_t:W - x_t].set(1.0)
    border_mask = border_mask.reshape(-1)
    p = prob * border_mask
    p = p / p.sum()
    p = jax.lax.stop_gradient(p)
    skey = jax.random.key(42)
    idx = jax.random.choice(skey, prob.shape[0], shape=(_NUM_POINTS,),
                            replace=True, p=p)
    y = idx // W
    x = idx % W
    return y, x


_SC_MESH = plsc.VectorSubcoreMesh(core_axis_name="c", subcore_axis_name="s")


_QROWS = 56
_NUNITS = 12


@functools.partial(
    pl.kernel,
    mesh=_SC_MESH,
    out_type=jax.ShapeDtypeStruct((_NUM_POINTS * 3 * _CROP, _CROP),
                                  jnp.float32),
    scratch_types=[
        [pltpu.VMEM((_QROWS,), jnp.int32) for _ in range(_NUNITS)],
        [pltpu.VMEM((_QROWS, 512), jnp.float32) for _ in range(2)],
        [pltpu.VMEM((_QROWS, _CROP), jnp.float32) for _ in range(2)],
        pltpu.VMEM((16,), jnp.int32),
        pltpu.SemaphoreType.DMA,
        pltpu.SemaphoreType.DMA,
    ],
    compiler_params=pltpu.CompilerParams(needs_layout_passes=False),
)
def _sc_crop_kernel(rows_hbm, idx_hbm, left_hbm, out_hbm,
                    idx_vs, g_vs, l_vs, lw_v, sem_g, sem_o):
    wid = lax.axis_index("s") * 2 + lax.axis_index("c")

    for u in range(_NUNITS):
        pltpu.sync_copy(
            idx_hbm.at[pl.ds((wid * _NUNITS + u) * _QROWS, _QROWS)],
            idx_vs[u])
    pltpu.sync_copy(left_hbm.at[pl.ds(wid * 16, 16)], lw_v)
    lane = lax.iota(jnp.int32, 16)
    lv = lax.reduce_sum_p.bind(
        jnp.where(lane == 0, lw_v[...], 0), axes=(0,))
    srcs = [lane + lv + 16 * j for j in range(14)]
    dsts = [lane + 16 * j for j in range(14)]

    def unit_out_row(u):
        c, q = u // 4, u % 4
        return (wid * 3 + c) * _CROP + q * _QROWS

    gcp = [None, None]
    ocp = [None, None]
    gcp[0] = pltpu.async_copy(rows_hbm.at[idx_vs[0]], g_vs[0], sem_g)
    for u in range(_NUNITS):
        b = u % 2
        gcp[b].wait()
        if u + 1 < _NUNITS:
            gcp[1 - b] = pltpu.async_copy(
                rows_hbm.at[idx_vs[u + 1]], g_vs[1 - b], sem_g)
        if ocp[b] is not None:
            ocp[b].wait()
        g_v, l_v = g_vs[b], l_vs[b]

        def body(i, carry):
            for r in range(4):
                row_idx = jnp.full((16,), 4 * i + r, dtype=jnp.int32)
                for j in range(14):
                    chunk = plsc.load_gather(g_v, [row_idx, srcs[j]])
                    plsc.store_scatter(l_v, [row_idx, dsts[j]], chunk)
            return carry

        lax.fori_loop(0, _QROWS // 4, body, 0)
        ocp[b] = pltpu.async_copy(
            l_v, out_hbm.at[pl.ds(unit_out_row(u), _QROWS), :], sem_o)
    for b in range(2):
        if ocp[b] is not None:
            ocp[b].wait()


def kernel(img, salience_map):
    y, x = _sample_yx(salience_map)
    half = _CROP // 2
    top = (y - half).astype(jnp.int32)
    left = (x - half).astype(jnp.int32)
    C, H, W = img.shape

    rows = img.reshape(C * H, W)
    cc = jnp.arange(3, dtype=jnp.int32)[None, :, None, None] * H
    qq = jnp.arange(4, dtype=jnp.int32)[None, None, :, None] * _QROWS
    ii = jnp.arange(_QROWS, dtype=jnp.int32)[None, None, None, :]
    idx = (top[:, None, None, None] + cc + qq + ii).reshape(-1)
    lpad = jnp.zeros((_NUM_POINTS, 16), jnp.int32).at[:, 0].set(left)

    out = _sc_crop_kernel(rows, idx, lpad.reshape(-1))
    return out.reshape(_NUM_POINTS, C, _CROP, _CROP)


# speedup vs baseline: 1.2368x; 1.0249x over previous
import functools

import jax
import jax.numpy as jnp
from jax import lax
from jax.experimental import pallas as pl
from jax.experimental.pallas import tpu as pltpu
from jax.experimental.pallas import tpu_sc as plsc

_NUM_POINTS = 32
_CROP = 224
_THRESHOLD = 0.15
_HALF_ROWS = _CROP // 2


def _sample_yx(salience_map):
    H, W = salience_map.shape
    prob = salience_map.reshape(-1)
    y_t = max(_CROP // 2, int(_THRESHOLD * H))
    x_t = max(_CROP // 2, int(_THRESHOLD * H))
    border_mask = jnp.zeros((H, W), dtype=salience_map.dtype)
    border_mask = border_mask.at[y_t:H - y_t, x_t:W - x_t].set(1.0)
    border_mask = border_mask.reshape(-1)
    p = prob * border_mask
    p = p / p.sum()
    p = jax.lax.stop_gradient(p)
    skey = jax.random.key(42)
    p_cuml = jnp.cumsum(p)
    rq = p_cuml[-1] * (1 - jax.random.uniform(skey, (_NUM_POINTS,),
                                              dtype=p_cuml.dtype))
    idx = jnp.searchsorted(p_cuml, rq, method='compare_all').astype(jnp.int32)
    y = idx // W
    x = idx % W
    return y, x


_SC_MESH = plsc.VectorSubcoreMesh(core_axis_name="c", subcore_axis_name="s")


_QROWS = 56
_NUNITS = 12


@functools.partial(
    pl.kernel,
    mesh=_SC_MESH,
    out_type=jax.ShapeDtypeStruct((_NUM_POINTS * 3 * _CROP, _CROP),
                                  jnp.float32),
    scratch_types=[
        [pltpu.VMEM((_QROWS,), jnp.int32) for _ in range(_NUNITS)],
        [pltpu.VMEM((_QROWS, 512), jnp.float32) for _ in range(2)],
        [pltpu.VMEM((_QROWS, _CROP), jnp.float32) for _ in range(2)],
        pltpu.VMEM((16,), jnp.int32),
        pltpu.SemaphoreType.DMA,
        pltpu.SemaphoreType.DMA,
    ],
    compiler_params=pltpu.CompilerParams(needs_layout_passes=False),
)
def _sc_crop_kernel(rows_hbm, idx_hbm, left_hbm, out_hbm,
                    idx_vs, g_vs, l_vs, lw_v, sem_g, sem_o):
    wid = lax.axis_index("s") * 2 + lax.axis_index("c")

    for u in range(_NUNITS):
        pltpu.sync_copy(
            idx_hbm.at[pl.ds((wid * _NUNITS + u) * _QROWS, _QROWS)],
            idx_vs[u])
    pltpu.sync_copy(left_hbm.at[pl.ds(wid * 16, 16)], lw_v)
    lane = lax.iota(jnp.int32, 16)
    lv = lax.reduce_sum_p.bind(
        jnp.where(lane == 0, lw_v[...], 0), axes=(0,))
    srcs = [lane + lv + 16 * j for j in range(14)]
    dsts = [lane + 16 * j for j in range(14)]

    def unit_out_row(u):
        c, q = u // 4, u % 4
        return (wid * 3 + c) * _CROP + q * _QROWS

    gcp = [None, None]
    ocp = [None, None]
    gcp[0] = pltpu.async_copy(rows_hbm.at[idx_vs[0]], g_vs[0], sem_g)
    for u in range(_NUNITS):
        b = u % 2
        gcp[b].wait()
        if u + 1 < _NUNITS:
            gcp[1 - b] = pltpu.async_copy(
                rows_hbm.at[idx_vs[u + 1]], g_vs[1 - b], sem_g)
        if ocp[b] is not None:
            ocp[b].wait()
        g_v, l_v = g_vs[b], l_vs[b]

        def body(i, carry):
            for r in range(4):
                row_idx = jnp.full((16,), 4 * i + r, dtype=jnp.int32)
                for j in range(14):
                    chunk = plsc.load_gather(g_v, [row_idx, srcs[j]])
                    plsc.store_scatter(l_v, [row_idx, dsts[j]], chunk)
            return carry

        lax.fori_loop(0, _QROWS // 4, body, 0)
        ocp[b] = pltpu.async_copy(
            l_v, out_hbm.at[pl.ds(unit_out_row(u), _QROWS), :], sem_o)
    for b in range(2):
        if ocp[b] is not None:
            ocp[b].wait()


def kernel(img, salience_map):
    y, x = _sample_yx(salience_map)
    half = _CROP // 2
    top = (y - half).astype(jnp.int32)
    left = (x - half).astype(jnp.int32)
    C, H, W = img.shape

    rows = img.reshape(C * H, W)
    cc = jnp.arange(3, dtype=jnp.int32)[None, :, None, None] * H
    qq = jnp.arange(4, dtype=jnp.int32)[None, None, :, None] * _QROWS
    ii = jnp.arange(_QROWS, dtype=jnp.int32)[None, None, None, :]
    idx = (top[:, None, None, None] + cc + qq + ii).reshape(-1)
    lpad = jnp.zeros((_NUM_POINTS, 16), jnp.int32).at[:, 0].set(left)

    out = _sc_crop_kernel(rows, idx, lpad.reshape(-1))
    return out.reshape(_NUM_POINTS, C, _CROP, _CROP)


# speedup vs baseline: 1.2593x; 1.0182x over previous
import functools

import jax
import jax.numpy as jnp
from jax import lax
from jax.experimental import pallas as pl
from jax.experimental.pallas import tpu as pltpu
from jax.experimental.pallas import tpu_sc as plsc

_NUM_POINTS = 32
_CROP = 224
_THRESHOLD = 0.15
_HALF_ROWS = _CROP // 2


def _sample_yx(salience_map):
    H, W = salience_map.shape
    prob = salience_map.reshape(-1)
    y_t = max(_CROP // 2, int(_THRESHOLD * H))
    x_t = max(_CROP // 2, int(_THRESHOLD * H))
    border_mask = jnp.zeros((H, W), dtype=salience_map.dtype)
    border_mask = border_mask.at[y_t:H - y_t, x_t:W - x_t].set(1.0)
    border_mask = border_mask.reshape(-1)
    p = prob * border_mask
    p = p / p.sum()
    p = jax.lax.stop_gradient(p)
    skey = jax.random.key(42)
    p_cuml = jnp.cumsum(p)
    rq = p_cuml[-1] * (1 - jax.random.uniform(skey, (_NUM_POINTS,),
                                              dtype=p_cuml.dtype))
    idx = jnp.searchsorted(p_cuml, rq, method='compare_all').astype(jnp.int32)
    y = idx // W
    x = idx % W
    return y, x


_SC_MESH = plsc.VectorSubcoreMesh(core_axis_name="c", subcore_axis_name="s")


_QROWS = 56
_NUNITS = 12


@functools.partial(
    pl.kernel,
    mesh=_SC_MESH,
    out_type=jax.ShapeDtypeStruct((_NUM_POINTS * 3 * _CROP, _CROP),
                                  jnp.float32),
    scratch_types=[
        pltpu.VMEM((_NUNITS * _QROWS,), jnp.int32),
        [pltpu.VMEM((_QROWS, 512), jnp.float32) for _ in range(2)],
        [pltpu.VMEM((_QROWS, _CROP), jnp.float32) for _ in range(2)],
        pltpu.VMEM((16,), jnp.int32),
        pltpu.SemaphoreType.DMA,
        pltpu.SemaphoreType.DMA,
    ],
    compiler_params=pltpu.CompilerParams(needs_layout_passes=False),
)
def _sc_crop_kernel(rows_hbm, idx_hbm, left_hbm, out_hbm,
                    idx_v, g_vs, l_vs, lw_v, sem_g, sem_o):
    wid = lax.axis_index("s") * 2 + lax.axis_index("c")

    pltpu.sync_copy(
        idx_hbm.at[pl.ds(wid * _NUNITS * _QROWS, _NUNITS * _QROWS)], idx_v)
    idx_vs = [idx_v.at[pl.ds(u * _QROWS, _QROWS)] for u in range(_NUNITS)]
    pltpu.sync_copy(left_hbm.at[pl.ds(wid * 16, 16)], lw_v)
    lane = lax.iota(jnp.int32, 16)
    lv = lax.reduce_sum_p.bind(
        jnp.where(lane == 0, lw_v[...], 0), axes=(0,))
    srcs = [lane + lv + 16 * j for j in range(14)]
    dsts = [lane + 16 * j for j in range(14)]

    def unit_out_row(u):
        c, q = u // 4, u % 4
        return (wid * 3 + c) * _CROP + q * _QROWS

    gcp = [None, None]
    ocp = [None, None]
    gcp[0] = pltpu.async_copy(rows_hbm.at[idx_vs[0]], g_vs[0], sem_g)
    for u in range(_NUNITS):
        b = u % 2
        gcp[b].wait()
        if u + 1 < _NUNITS:
            gcp[1 - b] = pltpu.async_copy(
                rows_hbm.at[idx_vs[u + 1]], g_vs[1 - b], sem_g)
        if ocp[b] is not None:
            ocp[b].wait()
        g_v, l_v = g_vs[b], l_vs[b]

        def body(i, row_idx):
            for r in range(8):
                for j in range(14):
                    chunk = plsc.load_gather(g_v, [row_idx, srcs[j]])
                    plsc.store_scatter(l_v, [row_idx, dsts[j]], chunk)
                row_idx = row_idx + 1
            return row_idx

        lax.fori_loop(0, _QROWS // 8, body,
                      jnp.zeros((16,), dtype=jnp.int32))
        ocp[b] = pltpu.async_copy(
            l_v, out_hbm.at[pl.ds(unit_out_row(u), _QROWS), :], sem_o)
    for b in range(2):
        if ocp[b] is not None:
            ocp[b].wait()


def kernel(img, salience_map):
    y, x = _sample_yx(salience_map)
    half = _CROP // 2
    top = (y - half).astype(jnp.int32)
    left = (x - half).astype(jnp.int32)
    C, H, W = img.shape

    rows = img.reshape(C * H, W)
    cc = jnp.arange(3, dtype=jnp.int32)[None, :, None, None] * H
    qq = jnp.arange(4, dtype=jnp.int32)[None, None, :, None] * _QROWS
    ii = jnp.arange(_QROWS, dtype=jnp.int32)[None, None, None, :]
    idx = (top[:, None, None, None] + cc + qq + ii).reshape(-1)
    lpad = jnp.zeros((_NUM_POINTS, 16), jnp.int32).at[:, 0].set(left)

    out = _sc_crop_kernel(rows, idx, lpad.reshape(-1))
    return out.reshape(_NUM_POINTS, C, _CROP, _CROP)


# speedup vs baseline: 1.3464x; 1.0692x over previous
import functools

import jax
import jax.numpy as jnp
from jax import lax
from jax.experimental import pallas as pl
from jax.experimental.pallas import tpu as pltpu
from jax.experimental.pallas import tpu_sc as plsc

_NUM_POINTS = 32
_CROP = 224
_THRESHOLD = 0.15
_QROWS = 56
_N_SC = 8
_N_TC = _NUM_POINTS - _N_SC
_UPT = 3


def _sample_yx(salience_map):
    H, W = salience_map.shape
    prob = salience_map.reshape(-1)
    y_t = max(_CROP // 2, int(_THRESHOLD * H))
    x_t = max(_CROP // 2, int(_THRESHOLD * H))
    border_mask = jnp.zeros((H, W), dtype=salience_map.dtype)
    border_mask = border_mask.at[y_t:H - y_t, x_t:W - x_t].set(1.0)
    border_mask = border_mask.reshape(-1)
    p = prob * border_mask
    p = p / p.sum()
    p = jax.lax.stop_gradient(p)
    skey = jax.random.key(42)
    p_cuml = jnp.cumsum(p)
    rq = p_cuml[-1] * (1 - jax.random.uniform(skey, (_NUM_POINTS,),
                                              dtype=p_cuml.dtype))
    idx = jnp.searchsorted(p_cuml, rq, method='compare_all').astype(jnp.int32)
    y = idx // W
    x = idx % W
    return y, x


_SC_MESH = plsc.VectorSubcoreMesh(core_axis_name="c", subcore_axis_name="s")


@functools.partial(
    pl.kernel,
    mesh=_SC_MESH,
    out_type=jax.ShapeDtypeStruct((_N_SC * 3 * _CROP, _CROP), jnp.float32),
    scratch_types=[
        pltpu.VMEM((_UPT * _QROWS,), jnp.int32),
        [pltpu.VMEM((_QROWS, 512), jnp.float32) for _ in range(2)],
        [pltpu.VMEM((_QROWS, _CROP), jnp.float32) for _ in range(2)],
        pltpu.VMEM((16,), jnp.int32),
        pltpu.SemaphoreType.DMA,
        pltpu.SemaphoreType.DMA,
    ],
    compiler_params=pltpu.CompilerParams(needs_layout_passes=False),
)
def _sc_crop_kernel(rows_hbm, idx_hbm, left_hbm, out_hbm,
                    idx_v, g_vs, l_vs, lw_v, sem_g, sem_o):
    wid = lax.axis_index("s") * 2 + lax.axis_index("c")

    pltpu.sync_copy(
        idx_hbm.at[pl.ds(wid * _UPT * _QROWS, _UPT * _QROWS)], idx_v)
    idx_vs = [idx_v.at[pl.ds(u * _QROWS, _QROWS)] for u in range(_UPT)]
    pltpu.sync_copy(left_hbm.at[pl.ds(wid * 16, 16)], lw_v)
    lane = lax.iota(jnp.int32, 16)
    lv = lax.reduce_sum_p.bind(
        jnp.where(lane == 0, lw_v[...], 0), axes=(0,))
    srcs = [lane + lv + 16 * j for j in range(14)]
    dsts = [lane + 16 * j for j in range(14)]

    out_base = []
    for uu in range(_UPT):
        g = wid * _UPT + uu
        k = g // 12
        u = g - k * 12
        out_base.append((k * 3 + u // 4) * _CROP + (u % 4) * _QROWS)

    gcp = [None, None]
    ocp = [None, None]
    gcp[0] = pltpu.async_copy(rows_hbm.at[idx_vs[0]], g_vs[0], sem_g)
    for uu in range(_UPT):
        b = uu % 2
        gcp[b].wait()
        if uu + 1 < _UPT:
            gcp[1 - b] = pltpu.async_copy(
                rows_hbm.at[idx_vs[uu + 1]], g_vs[1 - b], sem_g)
        if ocp[b] is not None:
            ocp[b].wait()
        g_v, l_v = g_vs[b], l_vs[b]

        def body(i, row_idx):
            for r in range(8):
                for j in range(14):
                    chunk = plsc.load_gather(g_v, [row_idx, srcs[j]])
                    plsc.store_scatter(l_v, [row_idx, dsts[j]], chunk)
                row_idx = row_idx + 1
            return row_idx

        lax.fori_loop(0, _QROWS // 8, body,
                      jnp.zeros((16,), dtype=jnp.int32))
        ocp[b] = pltpu.async_copy(
            l_v, out_hbm.at[pl.ds(out_base[uu], _QROWS), :], sem_o)
    for b in range(2):
        if ocp[b] is not None:
            ocp[b].wait()


def _tc_crop_kernel(top_ref, left_ref, img_ref, out_ref):
    i = pl.program_id(0)
    t = top_ref[i]
    l = left_ref[i]
    a0 = t // 8
    dt = t - a0 * 8
    slab = img_ref[:, pl.ds(a0, 29), :, :]
    slab = slab.reshape(3, 232, 512)
    slab = pltpu.roll(slab, -l, axis=2)[:, :, :_CROP]
    out_ref[0] = jax.lax.switch(
        dt, [(lambda d: (lambda: slab[:, d:d + _CROP, :]))(d)
             for d in range(8)])


def kernel(img, salience_map):
    y, x = _sample_yx(salience_map)
    half = _CROP // 2
    top = (y - half).astype(jnp.int32)
    left = (x - half).astype(jnp.int32)
    C, H, W = img.shape

    top_sc = top[_N_TC:]
    left_sc = left[_N_TC:]
    rows = img.reshape(C * H, W)
    cc = jnp.arange(3, dtype=jnp.int32)[None, :, None, None] * H
    qq = jnp.arange(4, dtype=jnp.int32)[None, None, :, None] * _QROWS
    ii = jnp.arange(_QROWS, dtype=jnp.int32)[None, None, None, :]
    idx = (top_sc[:, None, None, None] + cc + qq + ii).reshape(-1)
    tile_crop = (jnp.arange(32, dtype=jnp.int32) * _UPT) // 12
    lpad = jnp.zeros((32, 16), jnp.int32).at[:, 0].set(left_sc[tile_crop])

    out_sc = _sc_crop_kernel(rows, idx, lpad.reshape(-1))

    out_tc = pl.pallas_call(
        _tc_crop_kernel,
        grid=(_N_TC,),
        in_specs=[
            pl.BlockSpec(memory_space=pltpu.SMEM),
            pl.BlockSpec(memory_space=pltpu.SMEM),
            pl.BlockSpec((C, H // 8, 8, W), lambda i: (0, 0, 0, 0)),
        ],
        out_specs=pl.BlockSpec((1, C, _CROP, _CROP), lambda i: (i, 0, 0, 0)),
        out_shape=jax.ShapeDtypeStruct((_N_TC, C, _CROP, _CROP), img.dtype),
    )(top[:_N_TC], left[:_N_TC], img.reshape(C, H // 8, 8, W))

    return jnp.concatenate(
        [out_tc, out_sc.reshape(_N_SC, C, _CROP, _CROP)], axis=0)


# speedup vs baseline: 1.3512x; 1.0035x over previous
import functools

import jax
import jax.numpy as jnp
from jax import lax
from jax.experimental import pallas as pl
from jax.experimental.pallas import tpu as pltpu
from jax.experimental.pallas import tpu_sc as plsc

_NUM_POINTS = 32
_CROP = 224
_THRESHOLD = 0.15
_QROWS = 56
_N_SC = 8
_N_TC = _NUM_POINTS - _N_SC
_UPT = 3


def _sample_yx(salience_map):
    H, W = salience_map.shape
    prob = salience_map.reshape(-1)
    y_t = max(_CROP // 2, int(_THRESHOLD * H))
    x_t = max(_CROP // 2, int(_THRESHOLD * H))
    border_mask = jnp.zeros((H, W), dtype=salience_map.dtype)
    border_mask = border_mask.at[y_t:H - y_t, x_t:W - x_t].set(1.0)
    border_mask = border_mask.reshape(-1)
    p = prob * border_mask
    p = p / p.sum()
    p = jax.lax.stop_gradient(p)
    skey = jax.random.key(42)
    p_cuml = jnp.cumsum(p)
    rq = p_cuml[-1] * (1 - jax.random.uniform(skey, (_NUM_POINTS,),
                                              dtype=p_cuml.dtype))
    idx = jnp.searchsorted(p_cuml, rq, method='compare_all').astype(jnp.int32)
    y = idx // W
    x = idx % W
    return y, x


_SC_MESH = plsc.VectorSubcoreMesh(core_axis_name="c", subcore_axis_name="s")


@functools.partial(
    pl.kernel,
    mesh=_SC_MESH,
    out_type=jax.ShapeDtypeStruct((_N_SC * 3 * _CROP, _CROP), jnp.float32),
    scratch_types=[
        pltpu.VMEM((_UPT * _QROWS,), jnp.int32),
        [pltpu.VMEM((_QROWS, 512), jnp.float32) for _ in range(2)],
        [pltpu.VMEM((_QROWS, _CROP), jnp.float32) for _ in range(2)],
        pltpu.VMEM((16,), jnp.int32),
        pltpu.SemaphoreType.DMA,
        pltpu.SemaphoreType.DMA,
    ],
    compiler_params=pltpu.CompilerParams(needs_layout_passes=False),
)
def _sc_crop_kernel(rows_hbm, idx_hbm, left_hbm, out_hbm,
                    idx_v, g_vs, l_vs, lw_v, sem_g, sem_o):
    wid = lax.axis_index("s") * 2 + lax.axis_index("c")

    pltpu.sync_copy(
        idx_hbm.at[pl.ds(wid * _UPT * _QROWS, _UPT * _QROWS)], idx_v)
    idx_vs = [idx_v.at[pl.ds(u * _QROWS, _QROWS)] for u in range(_UPT)]
    pltpu.sync_copy(left_hbm.at[pl.ds(wid * 16, 16)], lw_v)
    lane = lax.iota(jnp.int32, 16)
    lv = lax.reduce_sum_p.bind(
        jnp.where(lane == 0, lw_v[...], 0), axes=(0,))
    srcs = [lane + lv + 16 * j for j in range(14)]
    dsts = [lane + 16 * j for j in range(14)]

    out_base = []
    for uu in range(_UPT):
        g = wid * _UPT + uu
        k = g // 12
        u = g - k * 12
        out_base.append((k * 3 + u // 4) * _CROP + (u % 4) * _QROWS)

    gcp = [None, None]
    ocp = [None, None]
    gcp[0] = pltpu.async_copy(rows_hbm.at[idx_vs[0]], g_vs[0], sem_g)
    for uu in range(_UPT):
        b = uu % 2
        gcp[b].wait()
        if uu + 1 < _UPT:
            gcp[1 - b] = pltpu.async_copy(
                rows_hbm.at[idx_vs[uu + 1]], g_vs[1 - b], sem_g)
        if ocp[b] is not None:
            ocp[b].wait()
        g_v, l_v = g_vs[b], l_vs[b]

        def body(i, row_idx):
            for r in range(8):
                for j in range(14):
                    chunk = plsc.load_gather(g_v, [row_idx, srcs[j]])
                    plsc.store_scatter(l_v, [row_idx, dsts[j]], chunk)
                row_idx = row_idx + 1
            return row_idx

        lax.fori_loop(0, _QROWS // 8, body,
                      jnp.zeros((16,), dtype=jnp.int32))
        ocp[b] = pltpu.async_copy(
            l_v, out_hbm.at[pl.ds(out_base[uu], _QROWS), :], sem_o)
    for b in range(2):
        if ocp[b] is not None:
            ocp[b].wait()


def _tc_crop_kernel(top_ref, left_ref, img_ref, out_ref):
    i = pl.program_id(0)
    t = top_ref[i]
    l = left_ref[i]
    a0 = t // 8
    dt = t - a0 * 8
    slab = img_ref[:, pl.ds(a0, 29), :, :]
    slab = slab.reshape(3, 232, 512)
    slab = pltpu.roll(slab, -l, axis=2)[:, :, :_CROP]
    out_ref[0] = jax.lax.switch(
        dt, [(lambda d: (lambda: slab[:, d:d + _CROP, :]))(d)
             for d in range(8)])


def kernel(img, salience_map):
    y, x = _sample_yx(salience_map)
    half = _CROP // 2
    top = (y - half).astype(jnp.int32)
    left = (x - half).astype(jnp.int32)
    C, H, W = img.shape

    top_sc = top[_N_TC:]
    left_sc = left[_N_TC:]
    rows = img.reshape(C * H, W)
    cc = jnp.arange(3, dtype=jnp.int32)[None, :, None, None] * H
    qq = jnp.arange(4, dtype=jnp.int32)[None, None, :, None] * _QROWS
    ii = jnp.arange(_QROWS, dtype=jnp.int32)[None, None, None, :]
    idx = (top_sc[:, None, None, None] + cc + qq + ii).reshape(-1)
    tile_crop = (jnp.arange(32, dtype=jnp.int32) * _UPT) // 12
    lpad = jnp.zeros((32, 16), jnp.int32).at[:, 0].set(left_sc[tile_crop])

    out_tc = pl.pallas_call(
        _tc_crop_kernel,
        grid=(_N_TC,),
        in_specs=[
            pl.BlockSpec(memory_space=pltpu.SMEM),
            pl.BlockSpec(memory_space=pltpu.SMEM),
            pl.BlockSpec((C, H // 8, 8, W), lambda i: (0, 0, 0, 0)),
        ],
        out_specs=pl.BlockSpec((1, C, _CROP, _CROP), lambda i: (i, 0, 0, 0)),
        out_shape=jax.ShapeDtypeStruct((_N_TC, C, _CROP, _CROP), img.dtype),
    )(top[:_N_TC], left[:_N_TC], img.reshape(C, H // 8, 8, W))

    out_sc = _sc_crop_kernel(rows, idx, lpad.reshape(-1))

    return jnp.concatenate(
        [out_tc, out_sc.reshape(_N_SC, C, _CROP, _CROP)], axis=0)
